# baseline (speedup 1.0000x reference)
import functools

import jax
import jax.numpy as jnp
from jax import lax
from jax.experimental import pallas as pl
from jax.experimental.pallas import tpu as pltpu
from jax.experimental.pallas import tpu_sc as plsc

E = 8
CAP = 320
T = 8192
D = 1024
DFF = 4096
BT = 512
NB = T // BT
NSLOT = E * CAP
NSLOT_PAD = 2880
EO_ROWS = NSLOT + CAP
FB = 2048
NJ = DFF // FB

NC, NS = 2, 16
NW = NC * NS


def _router_body(x_ref, wr_ref, br_ref, slot_ref, loss_ref, carry, psum,
                 triu_s):
    i = pl.program_id(0)

    @pl.when(i == 0)
    def _():
        carry[...] = jnp.zeros_like(carry)
        psum[...] = jnp.zeros_like(psum)
        sub = lax.broadcasted_iota(jnp.int32, (BT, BT), 0)
        lan = lax.broadcasted_iota(jnp.int32, (BT, BT), 1)
        triu_s[...] = (sub <= lan).astype(jnp.float32)

    x = x_ref[...]
    logits = lax.dot_general(wr_ref[...], x, (((0,), (1,)), ((), ())),
                             preferred_element_type=jnp.float32)
    logits = logits + br_ref[...]
    m = jnp.max(logits, axis=0, keepdims=True)
    ex = jnp.exp(logits - m)
    probs = ex / jnp.sum(ex, axis=0, keepdims=True)
    srow = lax.broadcasted_iota(jnp.int32, (E, BT), 0)
    eidx = jnp.min(jnp.where(logits == m, srow, E), axis=0, keepdims=True)
    onehot = (srow == eidx).astype(jnp.float32)

    cum = lax.dot_general(onehot, triu_s[...], (((1,), (0,)), ((), ())),
                          preferred_element_type=jnp.float32)
    pos = jnp.sum(onehot * (cum + carry[...]), axis=0, keepdims=True) - 1.0
    keep = pos < float(CAP)
    tok = (i * BT + lax.broadcasted_iota(jnp.int32, (1, BT), 1)) % CAP
    slot = jnp.where(keep, eidx.astype(jnp.float32) * float(CAP) + pos,
                     float(NSLOT) + tok.astype(jnp.float32))
    slot_ref[...] = slot.astype(jnp.int32).reshape(1, 1, BT)

    carry[...] = carry[...] + jnp.sum(onehot, axis=1, keepdims=True)
    psum[...] = psum[...] + jnp.sum(probs, axis=1, keepdims=True)
    loss_ref[...] = (float(E) / float(T * T)) * jnp.sum(
        carry[...] * psum[...], axis=0, keepdims=True)


def _router(x2d, wr, br_col):
    return pl.pallas_call(
        _router_body,
        grid=(NB,),
        in_specs=[
            pl.BlockSpec((BT, D), lambda i: (i, 0)),
            pl.BlockSpec((D, E), lambda i: (0, 0)),
            pl.BlockSpec((E, 1), lambda i: (0, 0)),
        ],
        out_specs=[
            pl.BlockSpec((1, 1, BT), lambda i: (i, 0, 0)),
            pl.BlockSpec((1, 1), lambda i: (0, 0)),
        ],
        out_shape=[
            jax.ShapeDtypeStruct((NB, 1, BT), jnp.int32),
            jax.ShapeDtypeStruct((1, 1), jnp.float32),
        ],
        scratch_shapes=[
            pltpu.VMEM((E, 1), jnp.float32),
            pltpu.VMEM((E, 1), jnp.float32),
            pltpu.VMEM((BT, BT), jnp.float32),
        ],
        compiler_params=pltpu.CompilerParams(
            dimension_semantics=("arbitrary",)),
    )(x2d, wr, br_col)


def _ffn_body(x_ref, w1_ref, b1_ref, w2_ref, b2_ref, o_ref):
    i = pl.program_id(0)
    j = pl.program_id(1)
    valid = i < E
    x = jnp.where(valid, x_ref[...], 0.0)
    xb = x.astype(jnp.bfloat16)
    w1 = w1_ref[0].astype(jnp.bfloat16)
    h = jnp.dot(xb, w1, preferred_element_type=jnp.float32) + b1_ref[0]
    hb = jnp.maximum(h, 0.0).astype(jnp.bfloat16)
    w2 = w2_ref[0].astype(jnp.bfloat16)
    acc = jnp.dot(hb, w2, preferred_element_type=jnp.float32)

    @pl.when(j == 0)
    def _():
        o_ref[...] = jnp.where(valid,
                               jnp.broadcast_to(b2_ref[0], (CAP, D)), 0.0)

    o_ref[...] += jnp.where(valid, acc, 0.0)


def _ffn(disp, w1, b1, w2, b2):
    ei = lambda i: jnp.minimum(i, E - 1)
    ej = lambda i, j: jnp.where(i == E, NJ - 1, j)
    return pl.pallas_call(
        _ffn_body,
        grid=(EO_ROWS // CAP, NJ),
        in_specs=[
            pl.BlockSpec((CAP, D), lambda i, j: (ei(i), 0)),
            pl.BlockSpec((1, D, FB), lambda i, j: (ei(i), 0, ej(i, j))),
            pl.BlockSpec((1, 1, FB), lambda i, j: (ei(i), 0, ej(i, j))),
            pl.BlockSpec((1, FB, D), lambda i, j: (ei(i), ej(i, j), 0)),
            pl.BlockSpec((1, 1, D), lambda i, j: (ei(i), 0, 0)),
        ],
        out_specs=pl.BlockSpec((CAP, D), lambda i, j: (i, 0)),
        out_shape=jax.ShapeDtypeStruct((EO_ROWS, D), jnp.float32),
        compiler_params=pltpu.CompilerParams(
            dimension_semantics=("arbitrary", "arbitrary"),
            vmem_limit_bytes=100 * 1024 * 1024),
    )(disp, w1, b1.reshape(E, 1, DFF), w2, b2.reshape(E, 1, D))


@functools.cache
def _mesh():
    return plsc.VectorSubcoreMesh(core_axis_name="c", subcore_axis_name="s")


def _wid():
    return lax.axis_index("s") * NC + lax.axis_index("c")


RPW = NSLOT // NW


def _dispatch_body(x_hbm, slots_hbm, disp_hbm, slots_v, map_v, shared_map,
                   idx_v, rows_v, sem):
    c = lax.axis_index("c")
    s = lax.axis_index("s")

    @pl.when(s == 0)
    def _():
        pltpu.sync_copy(slots_hbm, slots_v)

        @pl.loop(0, NSLOT_PAD // 16)
        def _(k):
            map_v[pl.ds(k * 16, 16)] = jnp.zeros((16,), jnp.int32)

        @pl.loop(0, T // 16)
        def _(k):
            idx = slots_v[pl.ds(k * 16, 16)]
            vals = lax.iota(jnp.int32, 16) + k * 16
            plsc.store_scatter(map_v, [idx], vals)

        pltpu.sync_copy(map_v, shared_map)

    plsc.subcore_barrier()
    base = c * (NSLOT // NC) + s * RPW
    pltpu.sync_copy(shared_map.at[pl.ds(base, RPW)], idx_v)
    pltpu.async_copy(x_hbm.at[idx_v], rows_v, sem).wait()
    pltpu.sync_copy(rows_v, disp_hbm.at[pl.ds(base, RPW)])


def _dispatch(x2d, slots):
    return pl.kernel(
        _dispatch_body, mesh=_mesh(),
        out_type=jax.ShapeDtypeStruct((NSLOT, D), jnp.float32),
        scratch_types=[pltpu.VMEM((T,), jnp.int32),
                       pltpu.VMEM((NSLOT_PAD,), jnp.int32),
                       pltpu.VMEM_SHARED((NSLOT_PAD,), jnp.int32),
                       pltpu.VMEM((RPW,), jnp.int32),
                       pltpu.VMEM((RPW, D), jnp.float32),
                       pltpu.SemaphoreType.DMA],
        compiler_params=pltpu.CompilerParams(
            needs_layout_passes=False))(x2d, slots)


CPW = T // NW
CH = 32
NCH = CPW // CH


def _combine_body(eo_hbm, slots_hbm, out_hbm, idx_v, rows, g0, g1, w0, w1):
    base = _wid() * CPW
    pltpu.sync_copy(slots_hbm.at[pl.ds(base, CPW)], idx_v)
    gsem = (g0, g1)
    wsem = (w0, w1)

    def gstart(k, b):
        pltpu.async_copy(eo_hbm.at[idx_v.at[pl.ds(k * CH, CH)]],
                         rows.at[b], gsem[b])

    def gwait(b):
        pltpu.make_async_copy(eo_hbm.at[idx_v.at[pl.ds(0, CH)]],
                              rows.at[b], gsem[b]).wait()

    def wstart(k, b):
        pltpu.async_copy(rows.at[b], out_hbm.at[pl.ds(base + k * CH, CH)],
                         wsem[b])

    def wwait(b):
        pltpu.make_async_copy(rows.at[b], out_hbm.at[pl.ds(base, CH)],
                              wsem[b]).wait()

    gstart(0, 0)
    gstart(1, 1)

    @pl.loop(0, NCH // 2)
    def _(k):
        e = 2 * k
        gwait(0)
        wstart(e, 0)
        gwait(1)
        wstart(e + 1, 1)

        @pl.when(k < NCH // 2 - 1)
        def _():
            wwait(0)
            gstart(e + 2, 0)
            wwait(1)
            gstart(e + 3, 1)

    wwait(0)
    wwait(1)


def _combine(eo, slots):
    return pl.kernel(
        _combine_body, mesh=_mesh(),
        out_type=jax.ShapeDtypeStruct((T, D), jnp.float32),
        scratch_types=[pltpu.VMEM((CPW,), jnp.int32),
                       pltpu.VMEM((2, CH, D), jnp.float32),
                       pltpu.SemaphoreType.DMA,
                       pltpu.SemaphoreType.DMA,
                       pltpu.SemaphoreType.DMA,
                       pltpu.SemaphoreType.DMA])(eo, slots)


def kernel(x, Wr, br, W1, b1, W2, b2):
    x2d = x.reshape(T, D)
    slots3, loss = _router(x2d, Wr, br.reshape(E, 1))
    slots = slots3.reshape(T)
    disp = _dispatch(x2d, slots)
    eo = _ffn(disp, W1, b1, W2, b2)
    out2d = _combine(eo, slots)
    return out2d.reshape(x.shape), loss[0, 0]


# speedup vs baseline: 1.0080x; 1.0080x over previous
import functools

import jax
import jax.numpy as jnp
from jax import lax
from jax.experimental import pallas as pl
from jax.experimental.pallas import tpu as pltpu
from jax.experimental.pallas import tpu_sc as plsc

E = 8
CAP = 320
T = 8192
D = 1024
DFF = 4096
BT = 512
NB = T // BT
NSLOT = E * CAP
NSLOT_PAD = 2880
EO_ROWS = NSLOT + CAP
FB = 2048
NJ = DFF // FB

NC, NS = 2, 16
NW = NC * NS


def _router_body(x_ref, wr_ref, br_ref, slot_ref, loss_ref, carry, psum,
                 triu_s):
    i = pl.program_id(0)

    @pl.when(i == 0)
    def _():
        carry[...] = jnp.zeros_like(carry)
        psum[...] = jnp.zeros_like(psum)
        sub = lax.broadcasted_iota(jnp.int32, (BT, BT), 0)
        lan = lax.broadcasted_iota(jnp.int32, (BT, BT), 1)
        triu_s[...] = (sub <= lan).astype(jnp.float32)

    x = x_ref[...]
    logits = lax.dot_general(wr_ref[...], x, (((0,), (1,)), ((), ())),
                             preferred_element_type=jnp.float32)
    logits = logits + br_ref[...]
    m = jnp.max(logits, axis=0, keepdims=True)
    ex = jnp.exp(logits - m)
    probs = ex / jnp.sum(ex, axis=0, keepdims=True)
    srow = lax.broadcasted_iota(jnp.int32, (E, BT), 0)
    eidx = jnp.min(jnp.where(logits == m, srow, E), axis=0, keepdims=True)
    onehot = (srow == eidx).astype(jnp.float32)

    cum = lax.dot_general(onehot, triu_s[...], (((1,), (0,)), ((), ())),
                          preferred_element_type=jnp.float32)
    pos = jnp.sum(onehot * (cum + carry[...]), axis=0, keepdims=True) - 1.0
    keep = pos < float(CAP)
    tok = (i * BT + lax.broadcasted_iota(jnp.int32, (1, BT), 1)) % CAP
    slot = jnp.where(keep, eidx.astype(jnp.float32) * float(CAP) + pos,
                     float(NSLOT) + tok.astype(jnp.float32))
    slot_ref[...] = slot.astype(jnp.int32).reshape(1, 1, BT)

    carry[...] = carry[...] + jnp.sum(onehot, axis=1, keepdims=True)
    psum[...] = psum[...] + jnp.sum(probs, axis=1, keepdims=True)
    loss_ref[...] = (float(E) / float(T * T)) * jnp.sum(
        carry[...] * psum[...], axis=0, keepdims=True)


def _router(x2d, wr, br_col):
    return pl.pallas_call(
        _router_body,
        grid=(NB,),
        in_specs=[
            pl.BlockSpec((BT, D), lambda i: (i, 0)),
            pl.BlockSpec((D, E), lambda i: (0, 0)),
            pl.BlockSpec((E, 1), lambda i: (0, 0)),
        ],
        out_specs=[
            pl.BlockSpec((1, 1, BT), lambda i: (i, 0, 0)),
            pl.BlockSpec((1, 1), lambda i: (0, 0)),
        ],
        out_shape=[
            jax.ShapeDtypeStruct((NB, 1, BT), jnp.int32),
            jax.ShapeDtypeStruct((1, 1), jnp.float32),
        ],
        scratch_shapes=[
            pltpu.VMEM((E, 1), jnp.float32),
            pltpu.VMEM((E, 1), jnp.float32),
            pltpu.VMEM((BT, BT), jnp.float32),
        ],
        compiler_params=pltpu.CompilerParams(
            dimension_semantics=("arbitrary",)),
    )(x2d, wr, br_col)


def _ffn_body(x_ref, w1_ref, b1_ref, w2_ref, b2_ref, o_ref):
    i = pl.program_id(0)
    j = pl.program_id(1)
    valid = i < E
    x = jnp.where(valid, x_ref[...], 0.0)
    xb = x.astype(jnp.bfloat16)
    w1 = w1_ref[0].astype(jnp.bfloat16)
    h = jnp.dot(xb, w1, preferred_element_type=jnp.float32) + b1_ref[0]
    hb = jnp.maximum(h, 0.0).astype(jnp.bfloat16)
    w2 = w2_ref[0].astype(jnp.bfloat16)
    acc = jnp.dot(hb, w2, preferred_element_type=jnp.float32)

    @pl.when(j == 0)
    def _():
        o_ref[...] = jnp.where(valid,
                               jnp.broadcast_to(b2_ref[0], (CAP, D)), 0.0)

    o_ref[...] += jnp.where(valid, acc, 0.0)


def _ffn(disp, w1, b1, w2, b2):
    ei = lambda i: jnp.minimum(i, E - 1)
    ej = lambda i, j: jnp.where(i == E, NJ - 1, j)
    return pl.pallas_call(
        _ffn_body,
        grid=(EO_ROWS // CAP, NJ),
        in_specs=[
            pl.BlockSpec((CAP, D), lambda i, j: (ei(i), 0)),
            pl.BlockSpec((1, D, FB), lambda i, j: (ei(i), 0, ej(i, j))),
            pl.BlockSpec((1, 1, FB), lambda i, j: (ei(i), 0, ej(i, j))),
            pl.BlockSpec((1, FB, D), lambda i, j: (ei(i), ej(i, j), 0)),
            pl.BlockSpec((1, 1, D), lambda i, j: (ei(i), 0, 0)),
        ],
        out_specs=pl.BlockSpec((CAP, D), lambda i, j: (i, 0)),
        out_shape=jax.ShapeDtypeStruct((EO_ROWS, D), jnp.float32),
        compiler_params=pltpu.CompilerParams(
            dimension_semantics=("arbitrary", "arbitrary"),
            vmem_limit_bytes=100 * 1024 * 1024),
    )(disp, w1, b1.reshape(E, 1, DFF), w2, b2.reshape(E, 1, D))


@functools.cache
def _mesh():
    return plsc.VectorSubcoreMesh(core_axis_name="c", subcore_axis_name="s")


def _wid():
    return lax.axis_index("s") * NC + lax.axis_index("c")


RPW = NSLOT // NW


def _dispatch_body(x_hbm, slots_hbm, disp_hbm, slots_v, map_v, shared_map,
                   idx_v, rows_v, sem):
    c = lax.axis_index("c")
    s = lax.axis_index("s")

    @pl.when(s == 0)
    def _():
        pltpu.sync_copy(slots_hbm, slots_v)

        @pl.loop(0, NSLOT_PAD // 16)
        def _(k):
            map_v[pl.ds(k * 16, 16)] = jnp.zeros((16,), jnp.int32)

        @pl.loop(0, T // 16)
        def _(k):
            idx = slots_v[pl.ds(k * 16, 16)]
            vals = lax.iota(jnp.int32, 16) + k * 16
            plsc.store_scatter(map_v, [idx], vals)

        pltpu.sync_copy(map_v, shared_map)

    plsc.subcore_barrier()
    base = c * (NSLOT // NC) + s * RPW
    pltpu.sync_copy(shared_map.at[pl.ds(base, RPW)], idx_v)
    pltpu.async_copy(x_hbm.at[idx_v], rows_v, sem).wait()
    pltpu.sync_copy(rows_v, disp_hbm.at[pl.ds(base, RPW)])


def _dispatch(x2d, slots):
    return pl.kernel(
        _dispatch_body, mesh=_mesh(),
        out_type=jax.ShapeDtypeStruct((NSLOT, D), jnp.float32),
        scratch_types=[pltpu.VMEM((T,), jnp.int32),
                       pltpu.VMEM((NSLOT_PAD,), jnp.int32),
                       pltpu.VMEM_SHARED((NSLOT_PAD,), jnp.int32),
                       pltpu.VMEM((RPW,), jnp.int32),
                       pltpu.VMEM((RPW, D), jnp.float32),
                       pltpu.SemaphoreType.DMA],
        compiler_params=pltpu.CompilerParams(
            needs_layout_passes=False))(x2d, slots)


CPW = T // NW
CH = 64


def _combine_body(eo_hbm, slots_hbm, out_hbm, idx_v, rows_v, sem):
    base = _wid() * CPW
    pltpu.sync_copy(slots_hbm.at[pl.ds(base, CPW)], idx_v)

    @pl.loop(0, CPW // CH)
    def _(k):
        pltpu.async_copy(eo_hbm.at[idx_v.at[pl.ds(k * CH, CH)]],
                         rows_v, sem).wait()
        pltpu.sync_copy(rows_v, out_hbm.at[pl.ds(base + k * CH, CH)])


def _combine(eo, slots):
    return pl.kernel(
        _combine_body, mesh=_mesh(),
        out_type=jax.ShapeDtypeStruct((T, D), jnp.float32),
        scratch_types=[pltpu.VMEM((CPW,), jnp.int32),
                       pltpu.VMEM((CH, D), jnp.float32),
                       pltpu.SemaphoreType.DMA])(eo, slots)


def kernel(x, Wr, br, W1, b1, W2, b2):
    x2d = x.reshape(T, D)
    slots3, loss = _router(x2d, Wr, br.reshape(E, 1))
    slots = slots3.reshape(T)
    disp = _dispatch(x2d, slots)
    eo = _ffn(disp, W1, b1, W2, b2)
    out2d = _combine(eo, slots)
    return out2d.reshape(x.shape), loss[0, 0]
